# baseline (speedup 1.0000x reference)
import functools

import jax
import jax.numpy as jnp
from jax import lax
from jax.experimental import pallas as pl
from jax.experimental.pallas import tpu as pltpu
from jax.experimental.pallas import tpu_sc as plsc

NFEAT = 128
NHID = 16
SHID = 32
ALPHA = 0.2
NHEADS = 4
MP = 2
B = 128
N = 128
D = NHEADS * NHID

_NC = 2
_NS = 16
_NW = _NC * _NS
_CH = 128

BT_H = 32
BT_C = 16


def _sc_gather(emb_table, idx_flat):
    tot = idx_flat.shape[0]
    per_w = tot // _NW
    nch = per_w // _CH
    mesh = plsc.VectorSubcoreMesh(core_axis_name="c", subcore_axis_name="s")

    @functools.partial(
        pl.kernel,
        mesh=mesh,
        out_type=jax.ShapeDtypeStruct((tot, NFEAT), jnp.float32),
        scratch_types=[
            pltpu.VMEM((per_w,), jnp.int32),
            pltpu.VMEM((2, _CH, NFEAT), jnp.float32),
            pltpu.SemaphoreType.DMA,
            pltpu.SemaphoreType.DMA,
            pltpu.SemaphoreType.DMA,
            pltpu.SemaphoreType.DMA,
        ],
    )
    def gather_kernel(table_hbm, idx_hbm, out_hbm, idx_v, rows_v, gsem0,
                      gsem1, osem0, osem1):
        wid = lax.axis_index("s") * _NC + lax.axis_index("c")
        base = wid * per_w
        pltpu.sync_copy(idx_hbm.at[pl.ds(base, per_w)], idx_v)
        gsems = (gsem0, gsem1)
        osems = (osem0, osem1)

        def start_gather(c):
            return pltpu.async_copy(
                table_hbm.at[idx_v.at[pl.ds(c * _CH, _CH)]],
                rows_v.at[c % 2], gsems[c % 2])

        gat = {0: start_gather(0)}
        out = {}
        for c in range(nch):
            if c + 1 < nch:
                if c - 1 >= 0:
                    out[c - 1].wait()
                gat[c + 1] = start_gather(c + 1)
            gat[c].wait()
            out[c] = pltpu.async_copy(
                rows_v.at[c % 2], out_hbm.at[pl.ds(base + c * _CH, _CH)],
                osems[c % 2])
        if nch >= 2:
            out[nch - 2].wait()
        out[nch - 1].wait()

    return gather_kernel(emb_table, idx_flat)


def _hgat_body(seq_ref, adj_ref, Wpr_ref, WdstT_ref, Ws_ref, bs_ref,
               q_ref, z_ref, w_ref):
    ones_blk = jnp.ones((N, D), jnp.bfloat16)
    lane = lax.broadcasted_iota(jnp.int32, (N, 2 * D), 1)
    hid = (lane % D) // NHID
    wrows = []
    for bb in range(BT_H):
        seq = seq_ref[bb].astype(jnp.bfloat16)
        pr = jnp.dot(seq, Wpr_ref[...],
                     preferred_element_type=jnp.float32)
        ed_all = lax.dot_general(WdstT_ref[...], seq, (((1,), (1,)), ((), ())),
                                 preferred_element_type=jnp.float32)
        for m in range(MP):
            Wh = pr[:, m * 128:m * 128 + D].astype(jnp.bfloat16)
            adj_m = adj_ref[bb, m]
            lm = jnp.log2(adj_m + 1e-30)
            WhE = jnp.concatenate([Wh, ones_blk], axis=1)
            WB = jnp.concatenate(
                [jnp.where(hid == h, WhE, 0) for h in range(NHEADS)], axis=0)
            ps = []
            for h in range(NHEADS):
                e = pr[:, m * 128 + D + h:m * 128 + D + h + 1] \
                    + ed_all[m * NHEADS + h:m * NHEADS + h + 1, :]
                ps.append(jnp.exp2(jnp.maximum(e, ALPHA * e) + lm)
                          .astype(jnp.bfloat16))
            acc = jnp.dot(jnp.concatenate(ps, axis=1), WB,
                          preferred_element_type=jnp.float32)
            z = acc[:, :D] / acc[:, D:2 * D]
            z = jnp.where(z > 0, z, jnp.exp(z) - 1.0)
            z_ref[bb, m] = z
            s = jnp.tanh(jnp.dot(z.astype(jnp.bfloat16), Ws_ref[...],
                                 preferred_element_type=jnp.float32)
                         + bs_ref[...])
            sq = s * q_ref[...]
            cs = jnp.sum(sq, axis=0, keepdims=True)
            wrows.append((m, cs))

    rows = []
    for m in range(MP):
        acc = sum(cs for (mm, cs) in wrows if mm == m)
        rows.append(jnp.concatenate(
            [acc, jnp.zeros((1, 128 - SHID), jnp.float32)], axis=1))
    w_ref[0] = jnp.concatenate(
        rows + [jnp.zeros((8 - MP, 128), jnp.float32)], axis=0)


def _combine_body(sig, z_ref, w_ref, msk_ref, pW1_ref, pb1_ref, pW2_ref,
                  pb2_ref, out_ref, beta_ref):
    @pl.when(pl.program_id(0) == 0)
    def _():
        wflat = w_ref[...].reshape((B // BT_H) * 8, 128)
        srow = lax.broadcasted_iota(jnp.int32, ((B // BT_H) * 8, 128), 0) % 8
        w0 = jnp.sum(jnp.where(srow == 0, wflat, 0.0)) / (B * N)
        w1 = jnp.sum(jnp.where(srow == 1, wflat, 0.0)) / (B * N)
        mx = jnp.maximum(w0, w1)
        e0 = jnp.exp(w0 - mx)
        e1 = jnp.exp(w1 - mx)
        beta_ref[0] = e0 / (e0 + e1)
        beta_ref[1] = e1 / (e0 + e1)

    beta0 = beta_ref[0]
    beta1 = beta_ref[1]
    rows = []
    for bb in range(BT_C):
        hcomb = beta0 * z_ref[bb, 0] + beta1 * z_ref[bb, 1]
        mrow = msk_ref[bb]
        num = jnp.sum(hcomb * mrow, axis=0, keepdims=True)
        den = jnp.sum(mrow) + 1e-10
        r = num / den
        if sig:
            r = 1.0 / (1.0 + jnp.exp(-r))
        rows.append(r)
    R = jnp.concatenate(rows, axis=0)
    t = jnp.maximum(
        jnp.dot(R, pW1_ref[...], preferred_element_type=jnp.float32)
        + pb1_ref[...], 0.0)
    out_ref[...] = (jnp.dot(t, pW2_ref[...], preferred_element_type=jnp.float32)
                    + pb2_ref[...])


def _hgat_view(seq, adj, Wpr, WdstT, Ws, bs2, q2):
    return pl.pallas_call(
        _hgat_body,
        grid=(B // BT_H,),
        in_specs=[
            pl.BlockSpec((BT_H, N, NFEAT), lambda i: (i, 0, 0)),
            pl.BlockSpec((BT_H, MP, N, N), lambda i: (i, 0, 0, 0)),
            pl.BlockSpec((NFEAT, MP * 128), lambda i: (0, 0)),
            pl.BlockSpec((MP * NHEADS, NFEAT), lambda i: (0, 0)),
            pl.BlockSpec((D, SHID), lambda i: (0, 0)),
            pl.BlockSpec((1, SHID), lambda i: (0, 0)),
            pl.BlockSpec((1, SHID), lambda i: (0, 0)),
        ],
        out_specs=[
            pl.BlockSpec((BT_H, MP, N, D), lambda i: (i, 0, 0, 0)),
            pl.BlockSpec((1, 8, 128), lambda i: (i, 0, 0)),
        ],
        out_shape=[
            jax.ShapeDtypeStruct((B, MP, N, D), jnp.float32),
            jax.ShapeDtypeStruct((B // BT_H, 8, 128), jnp.float32),
        ],
    )(seq, adj, Wpr, WdstT, Ws, bs2, q2)


def _combine_view(sig, z, w, msk, pW1, pb1_2, pW2, pb2_2):
    return pl.pallas_call(
        functools.partial(_combine_body, sig),
        grid=(B // BT_C,),
        in_specs=[
            pl.BlockSpec((BT_C, MP, N, D), lambda i: (i, 0, 0, 0)),
            pl.BlockSpec((B // BT_H, 8, 128), lambda i: (0, 0, 0)),
            pl.BlockSpec((BT_C, N, 1), lambda i: (i, 0, 0)),
            pl.BlockSpec((D, D), lambda i: (0, 0)),
            pl.BlockSpec((1, D), lambda i: (0, 0)),
            pl.BlockSpec((D, D), lambda i: (0, 0)),
            pl.BlockSpec((1, D), lambda i: (0, 0)),
        ],
        out_specs=pl.BlockSpec((BT_C, D), lambda i: (i, 0)),
        out_shape=jax.ShapeDtypeStruct((B, D), jnp.float32),
        scratch_shapes=[pltpu.SMEM((2,), jnp.float32)],
    )(z, w, msk, pW1, pb1_2, pW2, pb2_2)


def kernel(items, items_aug, adjs, aug_adjs, msk, msk_aug, emb_table, W_gat,
           a_src, a_dst, Ws, bs, q, pW1, pb1, pW2, pb2):
    seq_aug = _sc_gather(
        emb_table, items_aug.reshape(-1).astype(jnp.int32)).reshape(B, N, NFEAT)
    seq_reg = _sc_gather(
        emb_table, items.reshape(-1).astype(jnp.int32)).reshape(B, N, NFEAT)

    Wg = jnp.transpose(W_gat, (0, 2, 1, 3)).reshape(MP, NFEAT, D)
    log2e = 1.4426950408889634
    Wsrc = jnp.einsum('mhfd,mhd->mfh', W_gat, a_src) * log2e
    pad = jnp.zeros((MP, NFEAT, 128 - D - NHEADS), jnp.float32)
    Wpr = jnp.concatenate([Wg, Wsrc, pad], axis=2)
    Wpr = jnp.transpose(Wpr, (1, 0, 2)).reshape(
        NFEAT, MP * 128).astype(jnp.bfloat16)
    WdstT = (jnp.einsum('mhfd,mhd->mhf', W_gat, a_dst) * log2e).reshape(
        MP * NHEADS, NFEAT).astype(jnp.bfloat16)
    bs2 = bs.reshape(1, SHID)
    q2 = q.reshape(1, SHID)
    pb1_2 = pb1.reshape(1, D)
    pb2_2 = pb2.reshape(1, D)

    Ws_bf = Ws.astype(jnp.bfloat16)
    z_aug, w_aug = _hgat_view(seq_aug, aug_adjs, Wpr, WdstT, Ws_bf, bs2, q2)
    z_reg, w_reg = _hgat_view(seq_reg, adjs, Wpr, WdstT, Ws_bf, bs2, q2)

    c = _combine_view(False, z_aug, w_aug, msk_aug.reshape(B, N, 1),
                      pW1, pb1_2, pW2, pb2_2)
    c0 = _combine_view(True, z_reg, w_reg, msk.reshape(B, N, 1),
                       pW1, pb1_2, pW2, pb2_2)
    return (c, c0)


# speedup vs baseline: 1.0090x; 1.0090x over previous
import functools

import jax
import jax.numpy as jnp
from jax import lax
from jax.experimental import pallas as pl
from jax.experimental.pallas import tpu as pltpu
from jax.experimental.pallas import tpu_sc as plsc

NFEAT = 128
NHID = 16
SHID = 32
ALPHA = 0.2
NHEADS = 4
MP = 2
B = 128
N = 128
D = NHEADS * NHID

_NC = 2
_NS = 16
_NW = _NC * _NS
_CH = 128

BT_H = 16
BT_C = 16


def _sc_gather(emb_table, idx_flat):
    tot = idx_flat.shape[0]
    per_w = tot // _NW
    nch = per_w // _CH
    mesh = plsc.VectorSubcoreMesh(core_axis_name="c", subcore_axis_name="s")

    @functools.partial(
        pl.kernel,
        mesh=mesh,
        out_type=jax.ShapeDtypeStruct((tot, NFEAT), jnp.float32),
        scratch_types=[
            pltpu.VMEM((per_w,), jnp.int32),
            pltpu.VMEM((2, _CH, NFEAT), jnp.float32),
            pltpu.SemaphoreType.DMA,
            pltpu.SemaphoreType.DMA,
            pltpu.SemaphoreType.DMA,
            pltpu.SemaphoreType.DMA,
        ],
    )
    def gather_kernel(table_hbm, idx_hbm, out_hbm, idx_v, rows_v, gsem0,
                      gsem1, osem0, osem1):
        wid = lax.axis_index("s") * _NC + lax.axis_index("c")
        base = wid * per_w
        pltpu.sync_copy(idx_hbm.at[pl.ds(base, per_w)], idx_v)
        gsems = (gsem0, gsem1)
        osems = (osem0, osem1)

        def start_gather(c):
            return pltpu.async_copy(
                table_hbm.at[idx_v.at[pl.ds(c * _CH, _CH)]],
                rows_v.at[c % 2], gsems[c % 2])

        gat = {0: start_gather(0)}
        out = {}
        for c in range(nch):
            if c + 1 < nch:
                if c - 1 >= 0:
                    out[c - 1].wait()
                gat[c + 1] = start_gather(c + 1)
            gat[c].wait()
            out[c] = pltpu.async_copy(
                rows_v.at[c % 2], out_hbm.at[pl.ds(base + c * _CH, _CH)],
                osems[c % 2])
        if nch >= 2:
            out[nch - 2].wait()
        out[nch - 1].wait()

    return gather_kernel(emb_table, idx_flat)


def _hgat_body(seq_ref, adj_ref, Wpr_ref, WdstT_ref, Ws_ref, bs_ref,
               q_ref, z_ref, w_ref):
    ones_blk = jnp.ones((N, D), jnp.bfloat16)
    lane = lax.broadcasted_iota(jnp.int32, (N, 2 * D), 1)
    hid = (lane % D) // NHID
    wrows = []
    for bb in range(BT_H):
        seq = seq_ref[bb].astype(jnp.bfloat16)
        pr = jnp.dot(seq, Wpr_ref[...],
                     preferred_element_type=jnp.float32)
        ed_all = lax.dot_general(WdstT_ref[...], seq, (((1,), (1,)), ((), ())),
                                 preferred_element_type=jnp.float32)
        for m in range(MP):
            Wh = pr[:, m * 128:m * 128 + D].astype(jnp.bfloat16)
            adj_m = adj_ref[bb, m]
            adje = adj_m + 1e-30
            WhE = jnp.concatenate([Wh, ones_blk], axis=1)
            WB = jnp.concatenate(
                [jnp.where(hid == h, WhE, 0) for h in range(NHEADS)], axis=0)
            ps = []
            for h in range(NHEADS):
                e = pr[:, m * 128 + D + h:m * 128 + D + h + 1] \
                    + ed_all[m * NHEADS + h:m * NHEADS + h + 1, :]
                ps.append((jnp.exp2(jnp.maximum(e, ALPHA * e)) * adje)
                          .astype(jnp.bfloat16))
            acc = jnp.dot(jnp.concatenate(ps, axis=1), WB,
                          preferred_element_type=jnp.float32)
            z = acc[:, :D] / acc[:, D:2 * D]
            z = jnp.where(z > 0, z, jnp.exp(z) - 1.0)
            z_ref[bb, m] = z
            s = jnp.tanh(jnp.dot(z.astype(jnp.bfloat16), Ws_ref[...],
                                 preferred_element_type=jnp.float32)
                         + bs_ref[...])
            sq = s * q_ref[...]
            cs = jnp.sum(sq, axis=0, keepdims=True)
            wrows.append((m, cs))

    rows = []
    for m in range(MP):
        acc = sum(cs for (mm, cs) in wrows if mm == m)
        rows.append(jnp.concatenate(
            [acc, jnp.zeros((1, 128 - SHID), jnp.float32)], axis=1))
    w_ref[0] = jnp.concatenate(
        rows + [jnp.zeros((8 - MP, 128), jnp.float32)], axis=0)


def _combine_body(sig, z_ref, w_ref, msk_ref, pW1_ref, pb1_ref, pW2_ref,
                  pb2_ref, out_ref, beta_ref):
    @pl.when(pl.program_id(0) == 0)
    def _():
        wflat = w_ref[...].reshape((B // BT_H) * 8, 128)
        srow = lax.broadcasted_iota(jnp.int32, ((B // BT_H) * 8, 128), 0) % 8
        w0 = jnp.sum(jnp.where(srow == 0, wflat, 0.0)) / (B * N)
        w1 = jnp.sum(jnp.where(srow == 1, wflat, 0.0)) / (B * N)
        mx = jnp.maximum(w0, w1)
        e0 = jnp.exp(w0 - mx)
        e1 = jnp.exp(w1 - mx)
        beta_ref[0] = e0 / (e0 + e1)
        beta_ref[1] = e1 / (e0 + e1)

    beta0 = beta_ref[0]
    beta1 = beta_ref[1]
    rows = []
    for bb in range(BT_C):
        hcomb = beta0 * z_ref[bb, 0] + beta1 * z_ref[bb, 1]
        mrow = msk_ref[bb]
        num = jnp.sum(hcomb * mrow, axis=0, keepdims=True)
        den = jnp.sum(mrow) + 1e-10
        r = num / den
        if sig:
            r = 1.0 / (1.0 + jnp.exp(-r))
        rows.append(r)
    R = jnp.concatenate(rows, axis=0)
    t = jnp.maximum(
        jnp.dot(R, pW1_ref[...], preferred_element_type=jnp.float32)
        + pb1_ref[...], 0.0)
    out_ref[...] = (jnp.dot(t, pW2_ref[...], preferred_element_type=jnp.float32)
                    + pb2_ref[...])


def _hgat_view(seq, adj, Wpr, WdstT, Ws, bs2, q2):
    return pl.pallas_call(
        _hgat_body,
        grid=(B // BT_H,),
        in_specs=[
            pl.BlockSpec((BT_H, N, NFEAT), lambda i: (i, 0, 0)),
            pl.BlockSpec((BT_H, MP, N, N), lambda i: (i, 0, 0, 0)),
            pl.BlockSpec((NFEAT, MP * 128), lambda i: (0, 0)),
            pl.BlockSpec((MP * NHEADS, NFEAT), lambda i: (0, 0)),
            pl.BlockSpec((D, SHID), lambda i: (0, 0)),
            pl.BlockSpec((1, SHID), lambda i: (0, 0)),
            pl.BlockSpec((1, SHID), lambda i: (0, 0)),
        ],
        out_specs=[
            pl.BlockSpec((BT_H, MP, N, D), lambda i: (i, 0, 0, 0)),
            pl.BlockSpec((1, 8, 128), lambda i: (i, 0, 0)),
        ],
        out_shape=[
            jax.ShapeDtypeStruct((B, MP, N, D), jnp.float32),
            jax.ShapeDtypeStruct((B // BT_H, 8, 128), jnp.float32),
        ],
    )(seq, adj, Wpr, WdstT, Ws, bs2, q2)


def _combine_view(sig, z, w, msk, pW1, pb1_2, pW2, pb2_2):
    return pl.pallas_call(
        functools.partial(_combine_body, sig),
        grid=(B // BT_C,),
        in_specs=[
            pl.BlockSpec((BT_C, MP, N, D), lambda i: (i, 0, 0, 0)),
            pl.BlockSpec((B // BT_H, 8, 128), lambda i: (0, 0, 0)),
            pl.BlockSpec((BT_C, N, 1), lambda i: (i, 0, 0)),
            pl.BlockSpec((D, D), lambda i: (0, 0)),
            pl.BlockSpec((1, D), lambda i: (0, 0)),
            pl.BlockSpec((D, D), lambda i: (0, 0)),
            pl.BlockSpec((1, D), lambda i: (0, 0)),
        ],
        out_specs=pl.BlockSpec((BT_C, D), lambda i: (i, 0)),
        out_shape=jax.ShapeDtypeStruct((B, D), jnp.float32),
        scratch_shapes=[pltpu.SMEM((2,), jnp.float32)],
    )(z, w, msk, pW1, pb1_2, pW2, pb2_2)


def kernel(items, items_aug, adjs, aug_adjs, msk, msk_aug, emb_table, W_gat,
           a_src, a_dst, Ws, bs, q, pW1, pb1, pW2, pb2):
    seq_aug = _sc_gather(
        emb_table, items_aug.reshape(-1).astype(jnp.int32)).reshape(B, N, NFEAT)
    seq_reg = _sc_gather(
        emb_table, items.reshape(-1).astype(jnp.int32)).reshape(B, N, NFEAT)

    Wg = jnp.transpose(W_gat, (0, 2, 1, 3)).reshape(MP, NFEAT, D)
    log2e = 1.4426950408889634
    Wsrc = jnp.einsum('mhfd,mhd->mfh', W_gat, a_src) * log2e
    pad = jnp.zeros((MP, NFEAT, 128 - D - NHEADS), jnp.float32)
    Wpr = jnp.concatenate([Wg, Wsrc, pad], axis=2)
    Wpr = jnp.transpose(Wpr, (1, 0, 2)).reshape(
        NFEAT, MP * 128).astype(jnp.bfloat16)
    WdstT = (jnp.einsum('mhfd,mhd->mhf', W_gat, a_dst) * log2e).reshape(
        MP * NHEADS, NFEAT).astype(jnp.bfloat16)
    bs2 = bs.reshape(1, SHID)
    q2 = q.reshape(1, SHID)
    pb1_2 = pb1.reshape(1, D)
    pb2_2 = pb2.reshape(1, D)

    Ws_bf = Ws.astype(jnp.bfloat16)
    z_aug, w_aug = _hgat_view(seq_aug, aug_adjs, Wpr, WdstT, Ws_bf, bs2, q2)
    z_reg, w_reg = _hgat_view(seq_reg, adjs, Wpr, WdstT, Ws_bf, bs2, q2)

    c = _combine_view(False, z_aug, w_aug, msk_aug.reshape(B, N, 1),
                      pW1, pb1_2, pW2, pb2_2)
    c0 = _combine_view(True, z_reg, w_reg, msk.reshape(B, N, 1),
                       pW1, pb1_2, pW2, pb2_2)
    return (c, c0)
